# baseline (speedup 1.0000x reference)
import functools

import jax
import jax.numpy as jnp
from jax import lax
from jax.experimental import pallas as pl
from jax.experimental.pallas import tpu as pltpu
from jax.experimental.pallas import tpu_sc as plsc

_NUM_CORES = 2
_NUM_SUBCORES = 16
_NW = _NUM_CORES * _NUM_SUBCORES

_CHUNK = 400


@functools.lru_cache(maxsize=None)
def _make_gather(V, D, B, chunk):
    per_w = B // _NW
    nchunk = per_w // chunk
    assert per_w * _NW == B and nchunk * chunk == per_w
    mesh = plsc.VectorSubcoreMesh(core_axis_name="c", subcore_axis_name="s")

    @functools.partial(
        pl.kernel,
        out_type=jax.ShapeDtypeStruct((B, D), jnp.float32),
        mesh=mesh,
        scratch_types=[
            pltpu.VMEM((per_w,), jnp.int32),
            pltpu.VMEM((chunk, D), jnp.float32),
            pltpu.VMEM((chunk, D), jnp.float32),
            pltpu.SemaphoreType.DMA,
            pltpu.SemaphoreType.DMA,
            pltpu.SemaphoreType.DMA,
            pltpu.SemaphoreType.DMA,
        ],
    )
    def gather(table_hbm, idx_hbm, out_hbm, idx_all, rows0, rows1,
               gsem0, gsem1, ssem0, ssem1):
        wid = lax.axis_index("s") * _NUM_CORES + lax.axis_index("c")
        base = wid * per_w
        rows = (rows0, rows1)
        gsem = (gsem0, gsem1)
        ssem = (ssem0, ssem1)

        pltpu.sync_copy(idx_hbm.at[pl.ds(base, per_w)], idx_all)

        gathers = [None, None]
        stores = [None, None]
        gathers[0] = pltpu.async_copy(
            table_hbm.at[idx_all.at[pl.ds(0, chunk)]], rows[0], gsem[0])
        for g in range(nchunk):
            b = g & 1
            if g + 1 < nchunk:
                ob = 1 - b
                if stores[ob] is not None:
                    stores[ob].wait()
                    stores[ob] = None
                gathers[ob] = pltpu.async_copy(
                    table_hbm.at[idx_all.at[pl.ds((g + 1) * chunk, chunk)]],
                    rows[ob], gsem[ob])
            gathers[b].wait()
            stores[b] = pltpu.async_copy(
                rows[b], out_hbm.at[pl.ds(base + g * chunk, chunk)], ssem[b])
        for s in stores:
            if s is not None:
                s.wait()

    return gather


def kernel(x, table):
    Bx, H = x.shape
    V, D = table.shape
    tot = Bx * H
    idx = x.reshape(tot)
    out = _make_gather(V, D, tot, _CHUNK)(table, idx)
    return out.reshape(Bx, H, D)


# speedup vs baseline: 1.0001x; 1.0001x over previous
import functools

import jax
import jax.numpy as jnp
from jax import lax
from jax.experimental import pallas as pl
from jax.experimental.pallas import tpu as pltpu
from jax.experimental.pallas import tpu_sc as plsc

_NUM_CORES = 2
_NUM_SUBCORES = 16
_NW = _NUM_CORES * _NUM_SUBCORES

_CHUNK = 400


@functools.lru_cache(maxsize=None)
def _make_gather(V, D, B, chunk):
    per_w = B // _NW
    nchunk = per_w // chunk
    assert per_w * _NW == B and nchunk * chunk == per_w
    mesh = plsc.VectorSubcoreMesh(core_axis_name="c", subcore_axis_name="s")

    @functools.partial(
        pl.kernel,
        out_type=jax.ShapeDtypeStruct((B, D), jnp.float32),
        mesh=mesh,
        compiler_params=pltpu.CompilerParams(use_tc_tiling_on_sc=True),
        scratch_types=[
            pltpu.VMEM((per_w,), jnp.int32),
            pltpu.VMEM((chunk, D), jnp.float32),
            pltpu.VMEM((chunk, D), jnp.float32),
            pltpu.SemaphoreType.DMA,
            pltpu.SemaphoreType.DMA,
            pltpu.SemaphoreType.DMA,
            pltpu.SemaphoreType.DMA,
        ],
    )
    def gather(table_hbm, idx_hbm, out_hbm, idx_all, rows0, rows1,
               gsem0, gsem1, ssem0, ssem1):
        wid = lax.axis_index("s") * _NUM_CORES + lax.axis_index("c")
        base = wid * per_w
        rows = (rows0, rows1)
        gsem = (gsem0, gsem1)
        ssem = (ssem0, ssem1)

        pltpu.sync_copy(idx_hbm.at[pl.ds(base, per_w)], idx_all)

        gathers = [None, None]
        stores = [None, None]
        gathers[0] = pltpu.async_copy(
            table_hbm.at[idx_all.at[pl.ds(0, chunk)]], rows[0], gsem[0])
        for g in range(nchunk):
            b = g & 1
            if g + 1 < nchunk:
                ob = 1 - b
                if stores[ob] is not None:
                    stores[ob].wait()
                    stores[ob] = None
                gathers[ob] = pltpu.async_copy(
                    table_hbm.at[idx_all.at[pl.ds((g + 1) * chunk, chunk)]],
                    rows[ob], gsem[ob])
            gathers[b].wait()
            stores[b] = pltpu.async_copy(
                rows[b], out_hbm.at[pl.ds(base + g * chunk, chunk)], ssem[b])
        for s in stores:
            if s is not None:
                s.wait()

    return gather


def kernel(x, table):
    Bx, H = x.shape
    V, D = table.shape
    tot = Bx * H
    idx = x.reshape(tot)
    out = _make_gather(V, D, tot, _CHUNK)(table, idx)
    return out.reshape(Bx, H, D)


# speedup vs baseline: 1.5482x; 1.5480x over previous
import functools

import jax
import jax.numpy as jnp
from jax import lax
from jax.experimental import pallas as pl
from jax.experimental.pallas import tpu as pltpu
from jax.experimental.pallas import tpu_sc as plsc

_NUM_CORES = 2
_NUM_SUBCORES = 16
_NW = _NUM_CORES * _NUM_SUBCORES

_BPC = 8


@functools.lru_cache(maxsize=None)
def _make_gather(V, D, Bx, H):
    tot = Bx * H
    per_w = tot // _NW
    bat_w = Bx // _NW
    chunk = _BPC * H
    nchunk = bat_w // _BPC
    assert per_w * _NW == tot and nchunk * _BPC == bat_w
    mesh = plsc.VectorSubcoreMesh(core_axis_name="c", subcore_axis_name="s")

    @functools.partial(
        pl.kernel,
        out_type=jax.ShapeDtypeStruct((Bx, H, D), jnp.float32),
        mesh=mesh,
        compiler_params=pltpu.CompilerParams(use_tc_tiling_on_sc=True),
        scratch_types=[
            pltpu.VMEM((per_w,), jnp.int32),
            pltpu.VMEM((chunk, D), jnp.float32),
            pltpu.VMEM((chunk, D), jnp.float32),
            pltpu.SemaphoreType.DMA,
            pltpu.SemaphoreType.DMA,
            pltpu.SemaphoreType.DMA,
            pltpu.SemaphoreType.DMA,
        ],
    )
    def gather(table_hbm, idx_hbm, out_hbm, idx_all, rows0, rows1,
               gsem0, gsem1, ssem0, ssem1):
        wid = lax.axis_index("s") * _NUM_CORES + lax.axis_index("c")
        base = wid * per_w
        bbase = wid * bat_w
        rows = (rows0, rows1)
        gsem = (gsem0, gsem1)
        ssem = (ssem0, ssem1)

        pltpu.sync_copy(idx_hbm.at[pl.ds(base, per_w)], idx_all)

        def fire_gather(g, b):
            return pltpu.async_copy(
                table_hbm.at[idx_all.at[pl.ds(g * chunk, chunk)]],
                rows[b], gsem[b])

        def fire_stores(g, b):
            b0 = bbase + g * _BPC
            return [
                pltpu.async_copy(
                    rows[b].at[pl.ds(j * H, H)], out_hbm.at[b0 + j], ssem[b])
                for j in range(_BPC)
            ]

        gathers = [fire_gather(0, 0), None]
        stores = [[], []]
        for g in range(nchunk):
            b = g & 1
            if g + 1 < nchunk:
                ob = 1 - b
                for s in stores[ob]:
                    s.wait()
                gathers[ob] = fire_gather(g + 1, ob)
            gathers[b].wait()
            stores[b] = fire_stores(g, b)
        for sl in stores:
            for s in sl:
                s.wait()

    return gather


def kernel(x, table):
    Bx, H = x.shape
    V, D = table.shape
    idx = x.reshape(Bx * H)
    return _make_gather(V, D, Bx, H)(table, idx)


# speedup vs baseline: 2.3044x; 1.4885x over previous
import functools

import jax
import jax.numpy as jnp
from jax import lax
from jax.experimental import pallas as pl
from jax.experimental.pallas import tpu as pltpu
from jax.experimental.pallas import tpu_sc as plsc

_NUM_CORES = 2
_NUM_SUBCORES = 16
_NW = _NUM_CORES * _NUM_SUBCORES

_CHUNK = 200
_NBUF = 4


@functools.lru_cache(maxsize=None)
def _make_gather(V, D, B, chunk, nbuf):
    per_w = B // _NW
    nchunk = per_w // chunk
    assert per_w * _NW == B and nchunk * chunk == per_w
    mesh = plsc.VectorSubcoreMesh(core_axis_name="c", subcore_axis_name="s")

    @functools.partial(
        pl.kernel,
        out_type=jax.ShapeDtypeStruct((B, D), jnp.float32),
        mesh=mesh,
        compiler_params=pltpu.CompilerParams(use_tc_tiling_on_sc=True),
        scratch_types=(
            [pltpu.VMEM((per_w,), jnp.int32)]
            + [pltpu.VMEM((chunk, D), jnp.float32) for _ in range(nbuf)]
            + [pltpu.SemaphoreType.DMA for _ in range(2 * nbuf)]
        ),
    )
    def gather(table_hbm, idx_hbm, out_hbm, idx_all, *bufs):
        rows = bufs[:nbuf]
        gsem = bufs[nbuf:2 * nbuf]
        ssem = bufs[2 * nbuf:]
        wid = lax.axis_index("s") * _NUM_CORES + lax.axis_index("c")
        base = wid * per_w

        pltpu.sync_copy(idx_hbm.at[pl.ds(base, per_w)], idx_all)

        def fire_gather(g, b):
            return pltpu.async_copy(
                table_hbm.at[idx_all.at[pl.ds(g * chunk, chunk)]],
                rows[b], gsem[b])

        def fire_store(g, b):
            return pltpu.async_copy(
                rows[b], out_hbm.at[pl.ds(base + g * chunk, chunk)], ssem[b])

        gathers = [None] * nbuf
        stores = [None] * nbuf
        for j in range(min(nbuf - 1, nchunk)):
            gathers[j] = fire_gather(j, j)
        for g in range(nchunk):
            b = g % nbuf
            ng = g + nbuf - 1
            if ng < nchunk:
                pb = ng % nbuf
                if stores[pb] is not None:
                    stores[pb].wait()
                gathers[pb] = fire_gather(ng, pb)
            gathers[b].wait()
            stores[b] = fire_store(g, b)
        for s in stores:
            if s is not None:
                s.wait()

    return gather


def kernel(x, table):
    Bx, H = x.shape
    V, D = table.shape
    tot = Bx * H
    idx = jnp.transpose(x).reshape(tot)
    out = _make_gather(V, D, tot, _CHUNK, _NBUF)(table, idx)
    return out.reshape(H, Bx, D).transpose(1, 0, 2)


# speedup vs baseline: 5.5939x; 2.4275x over previous
import functools

import jax
import jax.numpy as jnp
from jax import lax
from jax.experimental import pallas as pl
from jax.experimental.pallas import tpu as pltpu
from jax.experimental.pallas import tpu_sc as plsc

_NUM_CORES = 2
_NUM_SUBCORES = 16
_NW = _NUM_CORES * _NUM_SUBCORES

_CHUNK = 200
_NBUF = 4


@functools.lru_cache(maxsize=None)
def _make_gather(V, D, B, chunk, nbuf):
    per_w = B // _NW
    nchunk = per_w // chunk
    assert per_w * _NW == B and nchunk * chunk == per_w
    mesh = plsc.VectorSubcoreMesh(core_axis_name="c", subcore_axis_name="s")

    @functools.partial(
        pl.kernel,
        out_type=jax.ShapeDtypeStruct((B, D), jnp.float32),
        mesh=mesh,
        compiler_params=pltpu.CompilerParams(use_tc_tiling_on_sc=True),
        scratch_types=(
            [pltpu.VMEM((per_w,), jnp.int32),
             pltpu.VMEM_SHARED((V, D), jnp.float32)]
            + [pltpu.VMEM((chunk, D), jnp.float32) for _ in range(nbuf)]
            + [pltpu.SemaphoreType.DMA for _ in range(2 * nbuf)]
        ),
    )
    def gather(table_hbm, idx_hbm, out_hbm, idx_all, table_sh, *bufs):
        rows = bufs[:nbuf]
        gsem = bufs[nbuf:2 * nbuf]
        ssem = bufs[2 * nbuf:]
        sid = lax.axis_index("s")
        wid = sid * _NUM_CORES + lax.axis_index("c")
        base = wid * per_w

        @pl.when(sid == 0)
        def _():
            pltpu.sync_copy(table_hbm, table_sh)

        pltpu.sync_copy(idx_hbm.at[pl.ds(base, per_w)], idx_all)
        plsc.subcore_barrier()

        def fire_gather(g, b):
            return pltpu.async_copy(
                table_sh.at[idx_all.at[pl.ds(g * chunk, chunk)]],
                rows[b], gsem[b])

        def fire_store(g, b):
            return pltpu.async_copy(
                rows[b], out_hbm.at[pl.ds(base + g * chunk, chunk)], ssem[b])

        gathers = [None] * nbuf
        stores = [None] * nbuf
        for j in range(min(nbuf - 1, nchunk)):
            gathers[j] = fire_gather(j, j)
        for g in range(nchunk):
            b = g % nbuf
            ng = g + nbuf - 1
            if ng < nchunk:
                pb = ng % nbuf
                if stores[pb] is not None:
                    stores[pb].wait()
                gathers[pb] = fire_gather(ng, pb)
            gathers[b].wait()
            stores[b] = fire_store(g, b)
        for s in stores:
            if s is not None:
                s.wait()

    return gather


def kernel(x, table):
    Bx, H = x.shape
    V, D = table.shape
    tot = Bx * H
    idx = jnp.transpose(x).reshape(tot)
    out = _make_gather(V, D, tot, _CHUNK, _NBUF)(table, idx)
    return out.reshape(H, Bx, D).transpose(1, 0, 2)
